# baseline (speedup 1.0000x reference)
import jax
import jax.numpy as jnp
from jax import lax
from jax.experimental import pallas as pl
from jax.experimental.pallas import tpu as pltpu
from jax.experimental.pallas import tpu_sc as plsc

_B = 64
_S = 2048
_D = 1024


def _body(seq_hbm, tl_hbm, out_hbm, tl_s, sem):
    pltpu.sync_copy(tl_hbm, tl_s)

    for b in range(_B):
        t = tl_s[b]
        row = (t + (_S - 1)) & (_S - 1)
        pltpu.make_async_copy(seq_hbm.at[b, row], out_hbm.at[b], sem).start()
    pltpu.make_async_copy(seq_hbm.at[0, pl.ds(0, _B), :], out_hbm, sem).wait()


@jax.jit
def kernel(sequence_embedding, token_len):
    mesh = plsc.ScalarSubcoreMesh(axis_name="c", num_cores=1)
    out = pl.kernel(
        _body,
        out_type=jax.ShapeDtypeStruct((_B, _D), jnp.float32),
        mesh=mesh,
        scratch_types=[
            pltpu.SMEM((_B,), jnp.int32),
            pltpu.SemaphoreType.DMA,
        ],
    )(sequence_embedding, token_len)
    return out


# speedup vs baseline: 1.0215x; 1.0215x over previous
import jax
import jax.numpy as jnp
from jax import lax
from jax.experimental import pallas as pl
from jax.experimental.pallas import tpu as pltpu
from jax.experimental.pallas import tpu_sc as plsc

_B = 64
_S = 2048
_D = 1024


def _body(seq_hbm, tl_hbm, out_hbm, tl_s, sem):
    pltpu.sync_copy(tl_hbm, tl_s)

    def fire(i, carry):
        for j in range(8):
            b = i * 8 + j
            t = tl_s[b]
            row = (t + (_S - 1)) & (_S - 1)
            pltpu.make_async_copy(seq_hbm.at[b, row], out_hbm.at[b], sem).start()
        return carry

    lax.fori_loop(0, _B // 8, fire, 0)
    pltpu.make_async_copy(seq_hbm.at[0, pl.ds(0, _B), :], out_hbm, sem).wait()


@jax.jit
def kernel(sequence_embedding, token_len):
    mesh = plsc.ScalarSubcoreMesh(axis_name="c", num_cores=1)
    out = pl.kernel(
        _body,
        out_type=jax.ShapeDtypeStruct((_B, _D), jnp.float32),
        mesh=mesh,
        scratch_types=[
            pltpu.SMEM((_B,), jnp.int32),
            pltpu.SemaphoreType.DMA,
        ],
    )(sequence_embedding, token_len)
    return out


# speedup vs baseline: 1.0223x; 1.0007x over previous
import jax
import jax.numpy as jnp
from jax import lax
from jax.experimental import pallas as pl
from jax.experimental.pallas import tpu as pltpu
from jax.experimental.pallas import tpu_sc as plsc

_B = 64
_S = 2048
_D = 1024


def _body(seq_hbm, tl_hbm, out_hbm, tl_s, sem):
    pltpu.sync_copy(tl_hbm, tl_s)

    def fire(b, carry):
        t = tl_s[b]
        row = (t + (_S - 1)) & (_S - 1)
        pltpu.make_async_copy(seq_hbm.at[b, row], out_hbm.at[b], sem).start()
        return carry

    lax.fori_loop(0, _B, fire, 0)
    pltpu.make_async_copy(seq_hbm.at[0, pl.ds(0, _B), :], out_hbm, sem).wait()


@jax.jit
def kernel(sequence_embedding, token_len):
    mesh = plsc.ScalarSubcoreMesh(axis_name="c", num_cores=1)
    out = pl.kernel(
        _body,
        out_type=jax.ShapeDtypeStruct((_B, _D), jnp.float32),
        mesh=mesh,
        scratch_types=[
            pltpu.SMEM((_B,), jnp.int32),
            pltpu.SemaphoreType.DMA,
        ],
    )(sequence_embedding, token_len)
    return out
